# baseline (speedup 1.0000x reference)
import jax
import jax.numpy as jnp
from jax import lax
from jax.experimental import pallas as pl
from jax.experimental.pallas import tpu as pltpu
from jax.experimental.pallas import tpu_sc as plsc

_B, _S, _D = 64, 577, 768
_NW = 32
_BPW = _B // _NW
_NR = 64
_NFULL = _S // _NR
_REM_R0 = _NFULL * _NR
_REM = _S - _REM_R0


def _add_rows(xbuf, pbuf, nrows):
    def row_body(r, carry):
        for c in range(_D // 16):
            sl = pl.ds(c * 16, 16)
            xbuf[r, sl] = xbuf[r, sl] + pbuf[r, sl]
        return carry
    lax.fori_loop(0, nrows, row_body, 0)


def _sc_body(patch_hbm, pos_hbm, out_hbm, xbuf, pbuf, xrem, prem):
    wid = lax.axis_index("s") * 2 + lax.axis_index("c")
    b0 = wid * _BPW

    for k in range(_NFULL):
        r0 = k * _NR
        pltpu.sync_copy(pos_hbm.at[pl.ds(r0, _NR), :], pbuf)
        for bi in range(_BPW):
            b = b0 + bi
            pltpu.sync_copy(patch_hbm.at[b, pl.ds(r0, _NR), :], xbuf)
            _add_rows(xbuf, pbuf, _NR)
            pltpu.sync_copy(xbuf, out_hbm.at[b, pl.ds(r0, _NR), :])

    pltpu.sync_copy(pos_hbm.at[pl.ds(_REM_R0, _REM), :], prem)
    for bi in range(_BPW):
        b = b0 + bi
        pltpu.sync_copy(patch_hbm.at[b, pl.ds(_REM_R0, _REM), :], xrem)
        _add_rows(xrem, prem, _REM)
        pltpu.sync_copy(xrem, out_hbm.at[b, pl.ds(_REM_R0, _REM), :])


def _sc_encode(patch, pos_table):
    mesh = plsc.VectorSubcoreMesh(
        core_axis_name="c", subcore_axis_name="s", num_cores=2, num_subcores=16)
    k = pl.kernel(
        _sc_body,
        out_type=jax.ShapeDtypeStruct((_B, _S, _D), jnp.float32),
        mesh=mesh,
        scratch_types=[
            pltpu.VMEM((_NR, _D), jnp.float32),
            pltpu.VMEM((_NR, _D), jnp.float32),
            pltpu.VMEM((_REM, _D), jnp.float32),
            pltpu.VMEM((_REM, _D), jnp.float32),
        ],
        compiler_params=pltpu.CompilerParams(use_tc_tiling_on_sc=True),
    )
    return k(patch, pos_table)


def kernel(patch, pos_table):
    return (_sc_encode(patch, pos_table), pos_table)


# speedup vs baseline: 1.6120x; 1.6120x over previous
import jax
import jax.numpy as jnp
from jax import lax
from jax.experimental import pallas as pl
from jax.experimental.pallas import tpu as pltpu

_CH = 2
_NBUF = 4


def _body(patch_hbm, pos_hbm, out_hbm, pos_v, in_bufs, out_bufs,
          pos_sem, in_sems, out_sems):
    B = patch_hbm.shape[0]
    nchunk = B // _CH

    def in_copy(i, slot):
        return pltpu.make_async_copy(
            patch_hbm.at[pl.ds(i * _CH, _CH)], in_bufs.at[slot], in_sems.at[slot])

    def out_copy(i, slot):
        return pltpu.make_async_copy(
            out_bufs.at[slot], out_hbm.at[pl.ds(i * _CH, _CH)], out_sems.at[slot])

    pos_cp = pltpu.make_async_copy(pos_hbm, pos_v, pos_sem)
    pos_cp.start()
    for i in range(_NBUF):
        in_copy(i, i).start()
    pos_cp.wait()

    def step(i, carry):
        slot = lax.rem(i, _NBUF)
        in_copy(i, slot).wait()

        @pl.when(i >= _NBUF)
        def _():
            out_copy(i - _NBUF, slot).wait()

        out_bufs[slot] = in_bufs[slot] + pos_v[...]
        out_copy(i, slot).start()

        @pl.when(i + _NBUF < nchunk)
        def _():
            in_copy(i + _NBUF, slot).start()

        return carry

    lax.fori_loop(0, nchunk, step, 0)

    for j in range(_NBUF):
        i = nchunk - _NBUF + j
        out_copy(i, i % _NBUF).wait()


def kernel(patch, pos_table):
    B, S, D = patch.shape
    encoded = pl.pallas_call(
        _body,
        in_specs=[
            pl.BlockSpec(memory_space=pl.ANY),
            pl.BlockSpec(memory_space=pl.ANY),
        ],
        out_specs=pl.BlockSpec(memory_space=pl.ANY),
        out_shape=jax.ShapeDtypeStruct((B, S, D), patch.dtype),
        scratch_shapes=[
            pltpu.VMEM((S, D), patch.dtype),
            pltpu.VMEM((_NBUF, _CH, S, D), patch.dtype),
            pltpu.VMEM((_NBUF, _CH, S, D), patch.dtype),
            pltpu.SemaphoreType.DMA,
            pltpu.SemaphoreType.DMA((_NBUF,)),
            pltpu.SemaphoreType.DMA((_NBUF,)),
        ],
        compiler_params=pltpu.CompilerParams(
            vmem_limit_bytes=56 * 1024 * 1024,
            skip_device_barrier=True),
    )(patch, pos_table)
    return (encoded, pos_table)


# speedup vs baseline: 1.6130x; 1.0006x over previous
import jax
import jax.numpy as jnp
from jax import lax
from jax.experimental import pallas as pl
from jax.experimental.pallas import tpu as pltpu

_CH = 2
_NBUF = 4


def _body(patch_hbm, pos_hbm, out_hbm, pos_v, in_bufs, out_bufs,
          pos_sem, in_sems, out_sems):
    B = patch_hbm.shape[0]
    nchunk = B // _CH

    def in_copy(i, slot):
        return pltpu.make_async_copy(
            patch_hbm.at[pl.ds(i * _CH, _CH)], in_bufs.at[slot], in_sems.at[slot])

    def out_copy(i, slot):
        return pltpu.make_async_copy(
            out_bufs.at[slot], out_hbm.at[pl.ds(i * _CH, _CH)], out_sems.at[slot])

    pos_cp = pltpu.make_async_copy(pos_hbm, pos_v, pos_sem)
    pos_cp.start()
    for i in range(_NBUF):
        in_copy(i, i).start()
    pos_cp.wait()

    def step(i, carry):
        slot = lax.rem(i, _NBUF)
        in_copy(i, slot).wait()

        @pl.when(i >= _NBUF)
        def _():
            out_copy(i - _NBUF, slot).wait()

        out_bufs[slot] = in_bufs[slot] + pos_v[...]
        out_copy(i, slot).start()

        @pl.when(i + _NBUF < nchunk)
        def _():
            in_copy(i + _NBUF, slot).start()

        return carry

    lax.fori_loop(0, nchunk, step, 0)

    for j in range(_NBUF):
        i = nchunk - _NBUF + j
        out_copy(i, i % _NBUF).wait()


def kernel(patch, pos_table):
    B, S, D = patch.shape
    encoded = pl.pallas_call(
        _body,
        in_specs=[
            pl.BlockSpec(memory_space=pl.ANY),
            pl.BlockSpec(memory_space=pl.ANY),
        ],
        out_specs=pl.BlockSpec(memory_space=pl.ANY),
        out_shape=jax.ShapeDtypeStruct((B, S, D), patch.dtype),
        scratch_shapes=[
            pltpu.VMEM((S, D), patch.dtype),
            pltpu.VMEM((_NBUF, _CH, S, D), patch.dtype),
            pltpu.VMEM((_NBUF, _CH, S, D), patch.dtype),
            pltpu.SemaphoreType.DMA,
            pltpu.SemaphoreType.DMA((_NBUF,)),
            pltpu.SemaphoreType.DMA((_NBUF,)),
        ],
        compiler_params=pltpu.CompilerParams(
            vmem_limit_bytes=56 * 1024 * 1024),
    )(patch, pos_table)
    return (encoded, pos_table)
